# baseline (speedup 1.0000x reference)
import functools

import jax
import jax.numpy as jnp
from jax import lax
from jax.experimental import pallas as pl
from jax.experimental.pallas import tpu as pltpu
from jax.experimental.pallas import tpu_sc as plsc

N = 10000
D = 128
H = 16
O = 3
E = 320000

NTILE = 16
NPAD = 10240
RPT = NPAD // NTILE
CHUNK = 128

_mesh = plsc.VectorSubcoreMesh(core_axis_name="c", subcore_axis_name="s")
_sc_params = pltpu.CompilerParams(use_tc_tiling_on_sc=False,
                                  needs_layout_passes=False)


NBUF = 8
NWIN = E // CHUNK
AW = NWIN // 32
AREM = NWIN - 32 * AW
ASUP = (AW + NBUF) // NBUF
DW = NWIN // NTILE
DREM = NWIN - NTILE * DW
DSUP = (DW + NBUF) // NBUF


def _bf16r(x):
    i = plsc.bitcast(x, jnp.int32)
    r = i + jnp.int32(0x7FFF) + ((i >> 16) & jnp.int32(1))
    r = r & jnp.int32(-65536)
    return plsc.bitcast(r, jnp.float32)


def _rsqrt16(x):
    i = plsc.bitcast(x, jnp.int32)
    i = jnp.int32(0x5F3759DF) - (i >> 1)
    y = plsc.bitcast(i, jnp.float32)
    for _ in range(3):
        y = y * (1.5 - 0.5 * x * y * y)
    return y


@functools.partial(
    pl.kernel,
    out_type=[
        jax.ShapeDtypeStruct((2, NPAD, H), jnp.float32),
        jax.ShapeDtypeStruct((NPAD,), jnp.float32),
        jax.ShapeDtypeStruct((NPAD // 8, 128), jnp.float32),
    ],
    mesh=_mesh,
    compiler_params=_sc_params,
    scratch_types=[
        pltpu.VMEM((DW + 1, CHUNK), jnp.int32),
        pltpu.VMEM((AW + 1, CHUNK), jnp.int32),
        pltpu.VMEM((AW + 1, CHUNK), jnp.int32),
        pltpu.VMEM((NBUF, CHUNK, H), jnp.float32),
        pltpu.VMEM((RPT, H), jnp.float32),
        pltpu.VMEM((RPT // 8, 128), jnp.float32),
        pltpu.VMEM((RPT, H), jnp.float32),
        pltpu.VMEM((CHUNK,), jnp.float32),
        pltpu.VMEM((RPT,), jnp.float32),
        pltpu.VMEM((RPT,), jnp.float32),
        pltpu.VMEM_SHARED((NPAD,), jnp.float32),
        pltpu.VMEM_SHARED((NPAD, H), jnp.float32),
        pltpu.VMEM_SHARED((NPAD, H), jnp.float32),
        pltpu.SemaphoreType.DMA,
        pltpu.SemaphoreType.DMA,
        pltpu.SemaphoreType.DMA,
        pltpu.SemaphoreType.DMA,
        pltpu.SemaphoreType.DMA,
        pltpu.SemaphoreType.DMA,
        pltpu.SemaphoreType.DMA,
        pltpu.SemaphoreType.DMA,
        pltpu.SemaphoreType.DMA,
        pltpu.SemaphoreType.DMA,
        pltpu.SemaphoreType.DMA,
        pltpu.SemaphoreType.DMA,
        pltpu.SemaphoreType.DMA,
        pltpu.SemaphoreType.DMA,
        pltpu.SemaphoreType.DMA,
        pltpu.SemaphoreType.DMA,
    ],
)
def _layer1_kernel(h_hbm, ei_hbm, zeros_hbm,
                   agg_hbm, dinv_hbm, dinvpk_hbm,
                   degidx_v, src_v, dst_v, rows_v, h_v, dpk_v, g_v, ones_v,
                   deg_v, dinv_v, deg_sh, g_sh, agg_sh,
                   g0, g1, g2, g3, g4, g5, g6, g7,
                   t0, t1, t2, t3, t4, t5, t6, t7):
    gsem = (g0, g1, g2, g3, g4, g5, g6, g7)
    ssem = (t0, t1, t2, t3, t4, t5, t6, t7)
    c = lax.axis_index("c")
    s = lax.axis_index("s")
    w = c * NTILE + s
    rs = pl.ds(s * RPT, RPT)

    one = jnp.full((16,), 1.0, jnp.float32)
    for k in range(CHUNK // 16):
        ones_v[pl.ds(k * 16, 16)] = one
    for k in range(RPT // 16):
        deg_v[pl.ds(k * 16, 16)] = one
    pltpu.sync_copy(deg_v, deg_sh.at[rs])
    dbase = DW * s + jnp.minimum(s, DREM)
    dcnt = DW + jnp.where(s < DREM, 1, 0)
    abase = AW * w + jnp.minimum(w, AREM)
    acnt = AW + jnp.where(w < AREM, 1, 0)

    @pl.when(s < DREM)
    def _():
        pltpu.sync_copy(ei_hbm.at[1, pl.ds(dbase, DW + 1)], degidx_v)

    @pl.when(s >= DREM)
    def _():
        pltpu.sync_copy(ei_hbm.at[1, pl.ds(dbase, DW)],
                        degidx_v.at[pl.ds(0, DW)])

    @pl.when(w < AREM)
    def _():
        pltpu.sync_copy(ei_hbm.at[0, pl.ds(abase, AW + 1)], src_v)
        pltpu.sync_copy(ei_hbm.at[1, pl.ds(abase, AW + 1)], dst_v)

    @pl.when(w >= AREM)
    def _():
        pltpu.sync_copy(ei_hbm.at[0, pl.ds(abase, AW)],
                        src_v.at[pl.ds(0, AW)])
        pltpu.sync_copy(ei_hbm.at[1, pl.ds(abase, AW)],
                        dst_v.at[pl.ds(0, AW)])

    pltpu.sync_copy(h_hbm.at[rs], h_v)
    plsc.subcore_barrier()

    for k in range(NBUF):
        pltpu.async_copy(ones_v, deg_sh.at[degidx_v.at[k]], ssem[k], add=True)

    def dbody(g, carry):
        base = g * NBUF
        for k in range(NBUF):
            i = base + k

            @pl.when(i - NBUF < dcnt)
            def _():
                pltpu.make_async_copy(ones_v, deg_sh.at[degidx_v.at[k]],
                                      ssem[k]).wait()

            @pl.when(i < dcnt)
            def _():
                pltpu.async_copy(ones_v, deg_sh.at[degidx_v.at[i]],
                                 ssem[k], add=True)
        return carry

    lax.fori_loop(1, DSUP, dbody, 0)
    for k in range(NBUF):
        i0 = NBUF * (DSUP - 1) + k

        @pl.when(i0 < dcnt)
        def _():
            pltpu.make_async_copy(ones_v, deg_sh.at[degidx_v.at[k]],
                                  ssem[k]).wait()

    plsc.subcore_barrier()

    pltpu.sync_copy(deg_sh.at[rs], deg_v)
    for k in range(RPT // 16):
        sl = pl.ds(k * 16, 16)
        dinv_v[sl] = _rsqrt16(deg_v[sl])

    def sbody(k, carry):
        dv = dinv_v[pl.ds(k * 16, 16)]
        base = k * 16
        for j in range(16):
            dvj = jnp.broadcast_to(dv[j], (16,))
            g_v[base + j, :] = h_v[base + j, :] * dvj
            dpk_v[2 * k + j // 8, pl.ds(16 * (j % 8), 16)] = dvj
        return carry

    lax.fori_loop(0, RPT // 16, sbody, 0)
    pltpu.sync_copy(g_v, g_sh.at[rs])

    @pl.when(c == 0)
    def _():
        pltpu.sync_copy(g_v, agg_sh.at[rs])
        pltpu.sync_copy(dinv_v, dinv_hbm.at[rs])
        pltpu.sync_copy(dpk_v, dinvpk_hbm.at[pl.ds(s * (RPT // 8), RPT // 8)])

    @pl.when(c != 0)
    def _():
        pltpu.sync_copy(zeros_hbm.at[rs], agg_sh.at[rs])

    plsc.subcore_barrier()

    for k in range(NBUF):
        pltpu.async_copy(g_sh.at[src_v.at[k]], rows_v.at[k], gsem[k])

    def body(g, carry):
        base = g * NBUF
        for k in range(NBUF):
            i = base + k

            @pl.when(i < acnt)
            def _():
                pltpu.make_async_copy(g_sh.at[src_v.at[i]],
                                      rows_v.at[k], gsem[k]).wait()
                pltpu.async_copy(rows_v.at[k], agg_sh.at[dst_v.at[i]],
                                 ssem[k], add=True)
        for k in range(NBUF):
            nxt = base + NBUF + k

            @pl.when(nxt < acnt)
            def _():
                pltpu.make_async_copy(rows_v.at[k],
                                      agg_sh.at[dst_v.at[base + k]],
                                      ssem[k]).wait()
                pltpu.async_copy(g_sh.at[src_v.at[nxt]], rows_v.at[k],
                                 gsem[k])
        return carry

    lax.fori_loop(0, ASUP, body, 0)
    for k in range(NBUF):
        pltpu.make_async_copy(rows_v.at[k], agg_sh.at[dst_v.at[k]],
                              ssem[k]).wait()
    plsc.subcore_barrier()
    pltpu.sync_copy(agg_sh.at[rs], agg_hbm.at[c, rs])


@functools.partial(
    pl.kernel,
    out_type=jax.ShapeDtypeStruct((2, NPAD // 8, 128), jnp.float32),
    mesh=_mesh,
    compiler_params=_sc_params,
    scratch_types=[
        pltpu.VMEM((AW + 1, CHUNK), jnp.int32),
        pltpu.VMEM((AW + 1, CHUNK), jnp.int32),
        pltpu.VMEM((NBUF, CHUNK, H), jnp.float32),
        pltpu.VMEM((RPT, H), jnp.float32),
        pltpu.VMEM((RPT, H), jnp.float32),
        pltpu.VMEM((RPT, H), jnp.float32),
        pltpu.VMEM((RPT // 8, 128), jnp.float32),
        pltpu.VMEM((H, H), jnp.float32),
        pltpu.VMEM((H,), jnp.float32),
        pltpu.VMEM((RPT,), jnp.float32),
        pltpu.VMEM_SHARED((NPAD, H), jnp.float32),
        pltpu.VMEM_SHARED((NPAD, H), jnp.float32),
        pltpu.SemaphoreType.DMA,
        pltpu.SemaphoreType.DMA,
        pltpu.SemaphoreType.DMA,
        pltpu.SemaphoreType.DMA,
        pltpu.SemaphoreType.DMA,
        pltpu.SemaphoreType.DMA,
        pltpu.SemaphoreType.DMA,
        pltpu.SemaphoreType.DMA,
        pltpu.SemaphoreType.DMA,
        pltpu.SemaphoreType.DMA,
        pltpu.SemaphoreType.DMA,
        pltpu.SemaphoreType.DMA,
        pltpu.SemaphoreType.DMA,
        pltpu.SemaphoreType.DMA,
        pltpu.SemaphoreType.DMA,
        pltpu.SemaphoreType.DMA,
    ],
)
def _layer2_kernel(agg1_hbm, dinv_hbm, w2_hbm, b1_hbm, ei_hbm,
                   zeros_hbm, out_hbm,
                   src_v, dst_v, rows_v, p0_v, p1_v, g_v, rb_v, w2_v, b1_v,
                   dinv_v, g_sh, agg_sh,
                   g0, g1, g2, g3, g4, g5, g6, g7,
                   t0, t1, t2, t3, t4, t5, t6, t7):
    gsem = (g0, g1, g2, g3, g4, g5, g6, g7)
    ssem = (t0, t1, t2, t3, t4, t5, t6, t7)
    c = lax.axis_index("c")
    s = lax.axis_index("s")
    w = c * NTILE + s
    rs = pl.ds(s * RPT, RPT)

    pltpu.sync_copy(agg1_hbm.at[0, rs], p0_v)
    pltpu.sync_copy(agg1_hbm.at[1, rs], p1_v)
    pltpu.sync_copy(dinv_hbm.at[rs], dinv_v)
    pltpu.sync_copy(w2_hbm, w2_v)
    pltpu.sync_copy(b1_hbm, b1_v)
    abase = AW * w + jnp.minimum(w, AREM)
    acnt = AW + jnp.where(w < AREM, 1, 0)

    @pl.when(w < AREM)
    def _():
        pltpu.sync_copy(ei_hbm.at[0, pl.ds(abase, AW + 1)], src_v)
        pltpu.sync_copy(ei_hbm.at[1, pl.ds(abase, AW + 1)], dst_v)

    @pl.when(w >= AREM)
    def _():
        pltpu.sync_copy(ei_hbm.at[0, pl.ds(abase, AW)],
                        src_v.at[pl.ds(0, AW)])
        pltpu.sync_copy(ei_hbm.at[1, pl.ds(abase, AW)],
                        dst_v.at[pl.ds(0, AW)])

    w2r = [_bf16r(w2_v[f, :]) for f in range(H)]
    b1vec = b1_v[pl.ds(0, 16)]

    def cbody(k, carry):
        base = k * 16
        dv = dinv_v[pl.ds(base, 16)]
        for j in range(16):
            i = base + j
            a = p0_v[i, :] + p1_v[i, :]
            r = jnp.maximum(a * dv[j] + b1vec, 0.0)
            r = _bf16r(r)
            h2 = r[0] * w2r[0]
            for f in range(1, H):
                h2 = h2 + r[f] * w2r[f]
            g_v[i, :] = h2 * dv[j]
        return carry

    lax.fori_loop(0, RPT // 16, cbody, 0)
    pltpu.sync_copy(g_v, g_sh.at[rs])

    @pl.when(c == 0)
    def _():
        pltpu.sync_copy(g_v, agg_sh.at[rs])

    @pl.when(c != 0)
    def _():
        pltpu.sync_copy(zeros_hbm.at[rs], agg_sh.at[rs])

    plsc.subcore_barrier()

    for k in range(NBUF):
        pltpu.async_copy(g_sh.at[src_v.at[k]], rows_v.at[k], gsem[k])

    def body(g, carry):
        base = g * NBUF
        for k in range(NBUF):
            i = base + k

            @pl.when(i < acnt)
            def _():
                pltpu.make_async_copy(g_sh.at[src_v.at[i]],
                                      rows_v.at[k], gsem[k]).wait()
                pltpu.async_copy(rows_v.at[k], agg_sh.at[dst_v.at[i]],
                                 ssem[k], add=True)
        for k in range(NBUF):
            nxt = base + NBUF + k

            @pl.when(nxt < acnt)
            def _():
                pltpu.make_async_copy(rows_v.at[k],
                                      agg_sh.at[dst_v.at[base + k]],
                                      ssem[k]).wait()
                pltpu.async_copy(g_sh.at[src_v.at[nxt]], rows_v.at[k],
                                 gsem[k])
        return carry

    lax.fori_loop(0, ASUP, body, 0)
    for k in range(NBUF):
        pltpu.make_async_copy(rows_v.at[k], agg_sh.at[dst_v.at[k]],
                              ssem[k]).wait()
    plsc.subcore_barrier()
    pltpu.sync_copy(agg_sh.at[rs], p0_v)

    def rbody(k, carry):
        base = k * 16
        for j in range(16):
            rb_v[2 * k + j // 8, pl.ds(16 * (j % 8), 16)] = p0_v[base + j, :]
        return carry

    lax.fori_loop(0, RPT // 16, rbody, 0)
    pltpu.sync_copy(rb_v, out_hbm.at[c, pl.ds(s * (RPT // 8), RPT // 8)])


NPK = NPAD // 8


def _tc1_body(x_ref, w1_ref, h_ref):
    h_ref[...] = jnp.dot(x_ref[...], w1_ref[...],
                         preferred_element_type=jnp.float32)


def _tc3_body(agg_ref, dinvpk_ref, b2_ref, out_ref):
    a = agg_ref[0] + agg_ref[1]
    out_ref[...] = a * dinvpk_ref[...] + b2_ref[...]


def kernel(x, edge_index, W1, b1, W2, b2):
    f32 = jnp.float32
    ei3 = edge_index.astype(jnp.int32).reshape(2, NWIN, CHUNK)
    xpad = jnp.pad(x, ((0, NPAD - N), (0, 0)))
    w2pad = jnp.pad(W2, ((0, 0), (0, H - O)))
    b2t = jnp.tile(jnp.pad(b2, (0, H - O)), 8).reshape(1, 128)
    zeros = jnp.zeros((NPAD, H), f32)

    h1 = pl.pallas_call(
        _tc1_body,
        out_shape=jax.ShapeDtypeStruct((NPAD, H), f32),
    )(xpad, W1)

    agg1, dinv1, dinvpk = _layer1_kernel(h1, ei3, zeros)

    agg2 = _layer2_kernel(agg1, dinv1, w2pad, b1, ei3, zeros)

    out = pl.pallas_call(
        _tc3_body,
        out_shape=jax.ShapeDtypeStruct((NPK, 128), f32),
    )(agg2, dinvpk, b2t)

    return out.reshape(NPAD, H)[:N, :O]


# speedup vs baseline: 1.0559x; 1.0559x over previous
import functools

import jax
import jax.numpy as jnp
from jax import lax
from jax.experimental import pallas as pl
from jax.experimental.pallas import tpu as pltpu
from jax.experimental.pallas import tpu_sc as plsc

N = 10000
D = 128
H = 16
O = 3
E = 320000

NTILE = 16
NPAD = 10240
RPT = NPAD // NTILE
CHUNK = 128

_mesh = plsc.VectorSubcoreMesh(core_axis_name="c", subcore_axis_name="s")
_sc_params = pltpu.CompilerParams(use_tc_tiling_on_sc=False,
                                  needs_layout_passes=False)


NBUF = 8
NWIN = E // CHUNK
AW = NWIN // 32
AREM = NWIN - 32 * AW
ASUP = (AW + NBUF) // NBUF
DW = NWIN // NTILE
DREM = NWIN - NTILE * DW
DSUP = (DW + NBUF) // NBUF


def _bf16r(x):
    i = plsc.bitcast(x, jnp.int32)
    r = i + jnp.int32(0x7FFF) + ((i >> 16) & jnp.int32(1))
    r = r & jnp.int32(-65536)
    return plsc.bitcast(r, jnp.float32)


def _rsqrt16(x):
    i = plsc.bitcast(x, jnp.int32)
    i = jnp.int32(0x5F3759DF) - (i >> 1)
    y = plsc.bitcast(i, jnp.float32)
    for _ in range(3):
        y = y * (1.5 - 0.5 * x * y * y)
    return y


@functools.partial(
    pl.kernel,
    out_type=jax.ShapeDtypeStruct((NPAD,), jnp.float32),
    mesh=_mesh,
    compiler_params=_sc_params,
    scratch_types=[
        pltpu.VMEM((DW + 1, CHUNK), jnp.int32),
        pltpu.VMEM((CHUNK,), jnp.float32),
        pltpu.VMEM((RPT,), jnp.float32),
        pltpu.VMEM((RPT,), jnp.float32),
        pltpu.VMEM_SHARED((NPAD,), jnp.float32),
        pltpu.SemaphoreType.DMA,
        pltpu.SemaphoreType.DMA,
        pltpu.SemaphoreType.DMA,
        pltpu.SemaphoreType.DMA,
        pltpu.SemaphoreType.DMA,
        pltpu.SemaphoreType.DMA,
        pltpu.SemaphoreType.DMA,
        pltpu.SemaphoreType.DMA,
    ],
)
def _deg_kernel(ei_hbm, dinv_hbm,
                degidx_v, ones_v, deg_v, dinv_v, deg_sh,
                t0, t1, t2, t3, t4, t5, t6, t7):
    ssem = (t0, t1, t2, t3, t4, t5, t6, t7)
    c = lax.axis_index("c")
    s = lax.axis_index("s")
    rs = pl.ds(s * RPT, RPT)
    one = jnp.full((16,), 1.0, jnp.float32)
    for k in range(CHUNK // 16):
        ones_v[pl.ds(k * 16, 16)] = one
    for k in range(RPT // 16):
        deg_v[pl.ds(k * 16, 16)] = one
    pltpu.sync_copy(deg_v, deg_sh.at[rs])
    dbase = DW * s + jnp.minimum(s, DREM)
    dcnt = DW + jnp.where(s < DREM, 1, 0)

    @pl.when(s < DREM)
    def _():
        pltpu.sync_copy(ei_hbm.at[1, pl.ds(dbase, DW + 1)], degidx_v)

    @pl.when(s >= DREM)
    def _():
        pltpu.sync_copy(ei_hbm.at[1, pl.ds(dbase, DW)],
                        degidx_v.at[pl.ds(0, DW)])

    plsc.subcore_barrier()
    for k in range(NBUF):
        pltpu.async_copy(ones_v, deg_sh.at[degidx_v.at[k]], ssem[k], add=True)

    def dbody(g, carry):
        base = g * NBUF
        for k in range(NBUF):
            i = base + k

            @pl.when(i - NBUF < dcnt)
            def _():
                pltpu.make_async_copy(ones_v, deg_sh.at[degidx_v.at[k]],
                                      ssem[k]).wait()

            @pl.when(i < dcnt)
            def _():
                pltpu.async_copy(ones_v, deg_sh.at[degidx_v.at[i]],
                                 ssem[k], add=True)
        return carry

    lax.fori_loop(1, DSUP, dbody, 0)
    for k in range(NBUF):
        i0 = NBUF * (DSUP - 1) + k

        @pl.when(i0 < dcnt)
        def _():
            pltpu.make_async_copy(ones_v, deg_sh.at[degidx_v.at[k]],
                                  ssem[k]).wait()

    plsc.subcore_barrier()
    pltpu.sync_copy(deg_sh.at[rs], deg_v)
    for k in range(RPT // 16):
        sl = pl.ds(k * 16, 16)
        dinv_v[sl] = _rsqrt16(deg_v[sl])

    @pl.when(c == 0)
    def _():
        pltpu.sync_copy(dinv_v, dinv_hbm.at[rs])


@functools.partial(
    pl.kernel,
    out_type=[
        jax.ShapeDtypeStruct((2, NPAD, H), jnp.float32),
        jax.ShapeDtypeStruct((NPAD // 8, 128), jnp.float32),
    ],
    mesh=_mesh,
    compiler_params=_sc_params,
    scratch_types=[
        pltpu.VMEM((AW + 1, CHUNK), jnp.int32),
        pltpu.VMEM((AW + 1, CHUNK), jnp.int32),
        pltpu.VMEM((NBUF, CHUNK, H), jnp.float32),
        pltpu.VMEM((RPT, H), jnp.float32),
        pltpu.VMEM((RPT // 8, 128), jnp.float32),
        pltpu.VMEM((RPT, H), jnp.float32),
        pltpu.VMEM((RPT,), jnp.float32),
        pltpu.VMEM_SHARED((NPAD, H), jnp.float32),
        pltpu.VMEM_SHARED((NPAD, H), jnp.float32),
        pltpu.SemaphoreType.DMA,
        pltpu.SemaphoreType.DMA,
        pltpu.SemaphoreType.DMA,
        pltpu.SemaphoreType.DMA,
        pltpu.SemaphoreType.DMA,
        pltpu.SemaphoreType.DMA,
        pltpu.SemaphoreType.DMA,
        pltpu.SemaphoreType.DMA,
        pltpu.SemaphoreType.DMA,
        pltpu.SemaphoreType.DMA,
        pltpu.SemaphoreType.DMA,
        pltpu.SemaphoreType.DMA,
        pltpu.SemaphoreType.DMA,
        pltpu.SemaphoreType.DMA,
        pltpu.SemaphoreType.DMA,
        pltpu.SemaphoreType.DMA,
    ],
)
def _layer1_kernel(h_hbm, dinv_hbm, ei_hbm, zeros_hbm,
                   agg_hbm, dinvpk_hbm,
                   src_v, dst_v, rows_v, h_v, dpk_v, g_v,
                   dinv_v, g_sh, agg_sh,
                   g0, g1, g2, g3, g4, g5, g6, g7,
                   t0, t1, t2, t3, t4, t5, t6, t7):
    gsem = (g0, g1, g2, g3, g4, g5, g6, g7)
    ssem = (t0, t1, t2, t3, t4, t5, t6, t7)
    c = lax.axis_index("c")
    s = lax.axis_index("s")
    w = c * NTILE + s
    rs = pl.ds(s * RPT, RPT)

    abase = AW * w + jnp.minimum(w, AREM)
    acnt = AW + jnp.where(w < AREM, 1, 0)

    @pl.when(w < AREM)
    def _():
        pltpu.sync_copy(ei_hbm.at[0, pl.ds(abase, AW + 1)], src_v)
        pltpu.sync_copy(ei_hbm.at[1, pl.ds(abase, AW + 1)], dst_v)

    @pl.when(w >= AREM)
    def _():
        pltpu.sync_copy(ei_hbm.at[0, pl.ds(abase, AW)],
                        src_v.at[pl.ds(0, AW)])
        pltpu.sync_copy(ei_hbm.at[1, pl.ds(abase, AW)],
                        dst_v.at[pl.ds(0, AW)])

    pltpu.sync_copy(h_hbm.at[rs], h_v)
    pltpu.sync_copy(dinv_hbm.at[rs], dinv_v)
    plsc.subcore_barrier()

    def sbody(k, carry):
        dv = dinv_v[pl.ds(k * 16, 16)]
        base = k * 16
        for j in range(16):
            dvj = jnp.broadcast_to(dv[j], (16,))
            g_v[base + j, :] = h_v[base + j, :] * dvj
            dpk_v[2 * k + j // 8, pl.ds(16 * (j % 8), 16)] = dvj
        return carry

    lax.fori_loop(0, RPT // 16, sbody, 0)
    pltpu.sync_copy(g_v, g_sh.at[rs])

    @pl.when(c == 0)
    def _():
        pltpu.sync_copy(g_v, agg_sh.at[rs])
        pltpu.sync_copy(dpk_v, dinvpk_hbm.at[pl.ds(s * (RPT // 8), RPT // 8)])

    @pl.when(c != 0)
    def _():
        pltpu.sync_copy(zeros_hbm.at[rs], agg_sh.at[rs])

    plsc.subcore_barrier()

    for k in range(NBUF):
        pltpu.async_copy(g_sh.at[src_v.at[k]], rows_v.at[k], gsem[k])

    def body(g, carry):
        base = g * NBUF
        for k in range(NBUF):
            i = base + k

            @pl.when(i < acnt)
            def _():
                pltpu.make_async_copy(g_sh.at[src_v.at[i]],
                                      rows_v.at[k], gsem[k]).wait()
                pltpu.async_copy(rows_v.at[k], agg_sh.at[dst_v.at[i]],
                                 ssem[k], add=True)
        for k in range(NBUF):
            nxt = base + NBUF + k

            @pl.when(nxt < acnt)
            def _():
                pltpu.make_async_copy(rows_v.at[k],
                                      agg_sh.at[dst_v.at[base + k]],
                                      ssem[k]).wait()
                pltpu.async_copy(g_sh.at[src_v.at[nxt]], rows_v.at[k],
                                 gsem[k])
        return carry

    lax.fori_loop(0, ASUP, body, 0)
    for k in range(NBUF):
        pltpu.make_async_copy(rows_v.at[k], agg_sh.at[dst_v.at[k]],
                              ssem[k]).wait()
    plsc.subcore_barrier()
    pltpu.sync_copy(agg_sh.at[rs], agg_hbm.at[c, rs])


@functools.partial(
    pl.kernel,
    out_type=jax.ShapeDtypeStruct((2, NPAD // 8, 128), jnp.float32),
    mesh=_mesh,
    compiler_params=_sc_params,
    scratch_types=[
        pltpu.VMEM((AW + 1, CHUNK), jnp.int32),
        pltpu.VMEM((AW + 1, CHUNK), jnp.int32),
        pltpu.VMEM((NBUF, CHUNK, H), jnp.float32),
        pltpu.VMEM((RPT, H), jnp.float32),
        pltpu.VMEM((RPT, H), jnp.float32),
        pltpu.VMEM((RPT, H), jnp.float32),
        pltpu.VMEM((RPT // 8, 128), jnp.float32),
        pltpu.VMEM((H, H), jnp.float32),
        pltpu.VMEM((H,), jnp.float32),
        pltpu.VMEM((RPT,), jnp.float32),
        pltpu.VMEM_SHARED((NPAD, H), jnp.float32),
        pltpu.VMEM_SHARED((NPAD, H), jnp.float32),
        pltpu.SemaphoreType.DMA,
        pltpu.SemaphoreType.DMA,
        pltpu.SemaphoreType.DMA,
        pltpu.SemaphoreType.DMA,
        pltpu.SemaphoreType.DMA,
        pltpu.SemaphoreType.DMA,
        pltpu.SemaphoreType.DMA,
        pltpu.SemaphoreType.DMA,
        pltpu.SemaphoreType.DMA,
        pltpu.SemaphoreType.DMA,
        pltpu.SemaphoreType.DMA,
        pltpu.SemaphoreType.DMA,
        pltpu.SemaphoreType.DMA,
        pltpu.SemaphoreType.DMA,
        pltpu.SemaphoreType.DMA,
        pltpu.SemaphoreType.DMA,
    ],
)
def _layer2_kernel(agg1_hbm, dinv_hbm, w2_hbm, b1_hbm, ei_hbm,
                   zeros_hbm, out_hbm,
                   src_v, dst_v, rows_v, p0_v, p1_v, g_v, rb_v, w2_v, b1_v,
                   dinv_v, g_sh, agg_sh,
                   g0, g1, g2, g3, g4, g5, g6, g7,
                   t0, t1, t2, t3, t4, t5, t6, t7):
    gsem = (g0, g1, g2, g3, g4, g5, g6, g7)
    ssem = (t0, t1, t2, t3, t4, t5, t6, t7)
    c = lax.axis_index("c")
    s = lax.axis_index("s")
    w = c * NTILE + s
    rs = pl.ds(s * RPT, RPT)

    pltpu.sync_copy(agg1_hbm.at[0, rs], p0_v)
    pltpu.sync_copy(agg1_hbm.at[1, rs], p1_v)
    pltpu.sync_copy(dinv_hbm.at[rs], dinv_v)
    pltpu.sync_copy(w2_hbm, w2_v)
    pltpu.sync_copy(b1_hbm, b1_v)
    abase = AW * w + jnp.minimum(w, AREM)
    acnt = AW + jnp.where(w < AREM, 1, 0)

    @pl.when(w < AREM)
    def _():
        pltpu.sync_copy(ei_hbm.at[0, pl.ds(abase, AW + 1)], src_v)
        pltpu.sync_copy(ei_hbm.at[1, pl.ds(abase, AW + 1)], dst_v)

    @pl.when(w >= AREM)
    def _():
        pltpu.sync_copy(ei_hbm.at[0, pl.ds(abase, AW)],
                        src_v.at[pl.ds(0, AW)])
        pltpu.sync_copy(ei_hbm.at[1, pl.ds(abase, AW)],
                        dst_v.at[pl.ds(0, AW)])

    w2r = [_bf16r(w2_v[f, :]) for f in range(H)]
    b1vec = b1_v[pl.ds(0, 16)]

    def cbody(k, carry):
        base = k * 16
        dv = dinv_v[pl.ds(base, 16)]
        for j in range(16):
            i = base + j
            a = p0_v[i, :] + p1_v[i, :]
            r = jnp.maximum(a * dv[j] + b1vec, 0.0)
            r = _bf16r(r)
            h2 = r[0] * w2r[0]
            for f in range(1, H):
                h2 = h2 + r[f] * w2r[f]
            g_v[i, :] = h2 * dv[j]
        return carry

    lax.fori_loop(0, RPT // 16, cbody, 0)
    pltpu.sync_copy(g_v, g_sh.at[rs])

    @pl.when(c == 0)
    def _():
        pltpu.sync_copy(g_v, agg_sh.at[rs])

    @pl.when(c != 0)
    def _():
        pltpu.sync_copy(zeros_hbm.at[rs], agg_sh.at[rs])

    plsc.subcore_barrier()

    for k in range(NBUF):
        pltpu.async_copy(g_sh.at[src_v.at[k]], rows_v.at[k], gsem[k])

    def body(g, carry):
        base = g * NBUF
        for k in range(NBUF):
            i = base + k

            @pl.when(i < acnt)
            def _():
                pltpu.make_async_copy(g_sh.at[src_v.at[i]],
                                      rows_v.at[k], gsem[k]).wait()
                pltpu.async_copy(rows_v.at[k], agg_sh.at[dst_v.at[i]],
                                 ssem[k], add=True)
        for k in range(NBUF):
            nxt = base + NBUF + k

            @pl.when(nxt < acnt)
            def _():
                pltpu.make_async_copy(rows_v.at[k],
                                      agg_sh.at[dst_v.at[base + k]],
                                      ssem[k]).wait()
                pltpu.async_copy(g_sh.at[src_v.at[nxt]], rows_v.at[k],
                                 gsem[k])
        return carry

    lax.fori_loop(0, ASUP, body, 0)
    for k in range(NBUF):
        pltpu.make_async_copy(rows_v.at[k], agg_sh.at[dst_v.at[k]],
                              ssem[k]).wait()
    plsc.subcore_barrier()
    pltpu.sync_copy(agg_sh.at[rs], p0_v)

    def rbody(k, carry):
        base = k * 16
        for j in range(16):
            rb_v[2 * k + j // 8, pl.ds(16 * (j % 8), 16)] = p0_v[base + j, :]
        return carry

    lax.fori_loop(0, RPT // 16, rbody, 0)
    pltpu.sync_copy(rb_v, out_hbm.at[c, pl.ds(s * (RPT // 8), RPT // 8)])


NPK = NPAD // 8


def _tc1_body(x_ref, w1_ref, h_ref):
    h_ref[...] = jnp.dot(x_ref[...], w1_ref[...],
                         preferred_element_type=jnp.float32)


def _tc3_body(agg_ref, dinvpk_ref, b2_ref, out_ref):
    a = agg_ref[0] + agg_ref[1]
    out_ref[...] = a * dinvpk_ref[...] + b2_ref[...]


def kernel(x, edge_index, W1, b1, W2, b2):
    f32 = jnp.float32
    ei3 = edge_index.astype(jnp.int32).reshape(2, NWIN, CHUNK)
    xpad = jnp.pad(x, ((0, NPAD - N), (0, 0)))
    w2pad = jnp.pad(W2, ((0, 0), (0, H - O)))
    b2t = jnp.tile(jnp.pad(b2, (0, H - O)), 8).reshape(1, 128)
    zeros = jnp.zeros((NPAD, H), f32)

    h1 = pl.pallas_call(
        _tc1_body,
        out_shape=jax.ShapeDtypeStruct((NPAD, H), f32),
    )(xpad, W1)

    dinv1 = _deg_kernel(ei3)
    agg1, dinvpk = _layer1_kernel(h1, dinv1, ei3, zeros)

    agg2 = _layer2_kernel(agg1, dinv1, w2pad, b1, ei3, zeros)

    out = pl.pallas_call(
        _tc3_body,
        out_shape=jax.ShapeDtypeStruct((NPK, 128), f32),
    )(agg2, dinvpk, b2t)

    return out.reshape(NPAD, H)[:N, :O]
